# baseline (speedup 1.0000x reference)
import jax
import jax.numpy as jnp
from jax.experimental import pallas as pl

_BLOCK_ROWS = 8192


def _copy_set_row2(x_ref, o_ref):
    o_ref[...] = x_ref[...]

    @pl.when(pl.program_id(0) == 0)
    def _():
        o_ref[2, :] = jnp.full((o_ref.shape[1],), 9.0, dtype=o_ref.dtype)


def kernel(x):
    rows, cols = x.shape
    grid = rows // _BLOCK_ROWS
    return pl.pallas_call(
        _copy_set_row2,
        grid=(grid,),
        in_specs=[pl.BlockSpec((_BLOCK_ROWS, cols), lambda i: (i, 0))],
        out_specs=pl.BlockSpec((_BLOCK_ROWS, cols), lambda i: (i, 0)),
        out_shape=jax.ShapeDtypeStruct((rows, cols), x.dtype),
    )(x)
